# baseline (speedup 1.0000x reference)
import functools

import jax
import jax.numpy as jnp
from jax import lax
from jax.experimental import pallas as pl
from jax.experimental.pallas import tpu as pltpu
from jax.experimental.pallas import tpu_sc as plsc

_N = 10000
_D = 128
_H = 64
_E_RAW = 320000
_E2 = _E_RAW + _N
_CH = 256
_NC, _NS = 2, 16
_NW = _NC * _NS
_EP = ((_E2 + _NW * _CH - 1) // (_NW * _CH)) * (_NW * _CH)
_EPW = _EP // _NW
_NCHUNK = _EPW // _CH
_NPAD = 10240
_RPT = _NPAD // _NS
_BLKN = 512
_EB = 4096

_sc_mesh = plsc.VectorSubcoreMesh(
    core_axis_name="c", subcore_axis_name="s", num_cores=_NC, num_subcores=_NS)


def _node_tables_kernel(x_ref, pos_ref, win_ref, bin_ref, wq_ref, wk_ref,
                        wv_ref, wa1_ref, wp1_ref, t1_ref, tv_ref):
    x1 = jnp.maximum(
        jnp.dot(x_ref[...], win_ref[...], preferred_element_type=jnp.float32)
        + bin_ref[...], 0.0)
    qa = jnp.dot(jnp.dot(x1, wq_ref[...], preferred_element_type=jnp.float32),
                 wa1_ref[...], preferred_element_type=jnp.float32)
    ka = jnp.dot(jnp.dot(x1, wk_ref[...], preferred_element_type=jnp.float32),
                 wa1_ref[...], preferred_element_type=jnp.float32)
    pp = jnp.dot(pos_ref[...], wp1_ref[...], preferred_element_type=jnp.float32)
    v = jnp.dot(x1, wv_ref[...], preferred_element_type=jnp.float32)
    t1_ref[...] = jnp.concatenate([qa, pp], axis=1)
    tv_ref[...] = jnp.concatenate([ka, pp, v], axis=1)


def _edge_mlp_kernel(g_ref, vs_ref, bp1_ref, wp2_ref, bp2_ref, wa1_ref,
                     ba1_ref, wa2_ref, ba2_ref, ex_ref, m_ref):
    g = g_ref[...]
    hp = jnp.maximum(g[:, _H:] + bp1_ref[...], 0.0)
    delta = jnp.maximum(
        jnp.dot(hp, wp2_ref[...], preferred_element_type=jnp.float32)
        + bp2_ref[...], 0.0)
    da = jnp.dot(delta, wa1_ref[...], preferred_element_type=jnp.float32)
    h1 = jnp.maximum(g[:, :_H] + da + ba1_ref[...], 0.0)
    a = jnp.maximum(
        jnp.dot(h1, wa2_ref[...], preferred_element_type=jnp.float32)
        + ba2_ref[...], 0.0)
    ex = jnp.exp(a)
    ex_ref[...] = ex
    m_ref[...] = ex * (vs_ref[...] + delta)


def _out_kernel(nlo_ref, nhi_ref, dlo_ref, dhi_ref, wout_ref, bout_ref, o_ref):
    num = nlo_ref[...] + nhi_ref[...]
    den = dlo_ref[...] + dhi_ref[...]
    node = num / (den + 1e-16)
    o_ref[...] = jnp.maximum(
        jnp.dot(node, wout_ref[...], preferred_element_type=jnp.float32)
        + bout_ref[...], 0.0)


@functools.partial(
    pl.kernel,
    out_type=[
        jax.ShapeDtypeStruct((_EP, _D), jnp.float32),
        jax.ShapeDtypeStruct((_EP, _D), jnp.float32),
    ],
    mesh=_sc_mesh,
    scratch_types=[
        pltpu.VMEM((_CH,), jnp.int32),
        pltpu.VMEM((_CH,), jnp.int32),
        pltpu.VMEM((_CH, _D), jnp.float32),
        pltpu.VMEM((_CH, 2 * _D), jnp.float32),
        pltpu.SemaphoreType.DMA,
        pltpu.SemaphoreType.DMA,
        pltpu.SemaphoreType.DMA,
    ],
)
def _sc_gather(t1, tv, dstr, srcr, g, vs, di, si, b1, btv, s1, s2, s3):
    c = lax.axis_index("c")
    s = lax.axis_index("s")
    base0 = (s * _NC + c) * _EPW

    def chunk(i, carry):
        base = base0 + i * _CH
        ha = pltpu.async_copy(dstr.at[pl.ds(base, _CH)], di, s1)
        hb = pltpu.async_copy(srcr.at[pl.ds(base, _CH)], si, s2)
        ha.wait()
        hb.wait()
        h1 = pltpu.async_copy(t1.at[di], b1, s1)
        h2 = pltpu.async_copy(tv.at[si], btv, s2)
        h1.wait()
        h2.wait()

        def row(r, cr):
            for cc in range(_D // 16):
                sl = pl.ds(cc * 16, 16)
                b1[r, sl] = b1[r, sl] - btv[r, sl]
            return cr

        lax.fori_loop(0, _CH, row, 0)
        hw1 = pltpu.async_copy(b1, g.at[pl.ds(base, _CH)], s1)
        hw2 = pltpu.async_copy(btv.at[:, pl.ds(_D, _D)],
                               vs.at[pl.ds(base, _CH)], s3)
        hw1.wait()
        hw2.wait()
        return carry

    lax.fori_loop(0, _NCHUNK, chunk, 0)


def _zero_acc(buf, acc, s):
    def zrow(r, cr):
        for cc in range(_D // 16):
            buf[r, pl.ds(cc * 16, 16)] = jnp.zeros((16,), jnp.float32)
        return cr

    lax.fori_loop(0, _CH, zrow, 0)
    off = 0
    for sz in (_CH, _CH, _RPT - 2 * _CH):
        pltpu.sync_copy(buf.at[pl.ds(0, sz)],
                        acc.at[pl.ds(s * _RPT + off, sz)])
        off += sz


@functools.partial(
    pl.kernel,
    out_type=[
        jax.ShapeDtypeStruct((2 * _NPAD, _D), jnp.float32),
        jax.ShapeDtypeStruct((2 * _NPAD, _D), jnp.float32),
    ],
    mesh=_sc_mesh,
    scratch_types=[
        pltpu.VMEM((_CH,), jnp.int32),
        pltpu.VMEM((_CH, _D), jnp.float32),
        pltpu.VMEM_SHARED((_NPAD, _D), jnp.float32),
        pltpu.SemaphoreType.DMA,
        pltpu.SemaphoreType.DMA,
    ],
)
def _sc_scatter(mr, exr, dstr, npart, dpart, di, bval, acc, s1, s2):
    c = lax.axis_index("c")
    s = lax.axis_index("s")
    base0 = (s * _NC + c) * _EPW

    for valr, part in ((mr, npart), (exr, dpart)):
        _zero_acc(bval, acc, s)
        plsc.subcore_barrier()

        def chunk(i, carry):
            base = base0 + i * _CH
            ha = pltpu.async_copy(dstr.at[pl.ds(base, _CH)], di, s1)
            hb = pltpu.async_copy(valr.at[pl.ds(base, _CH)], bval, s2)
            ha.wait()
            hb.wait()
            pltpu.sync_copy(bval, acc.at[di], add=True)
            return carry

        lax.fori_loop(0, _NCHUNK, chunk, 0)
        plsc.subcore_barrier()
        pltpu.sync_copy(acc.at[pl.ds(s * _RPT, _RPT)],
                        part.at[pl.ds(c * _NPAD + s * _RPT, _RPT)])
        plsc.subcore_barrier()


def _full(shape):
    return pl.BlockSpec(shape, lambda i: (0, 0))


def kernel(x, pos, edge_index, W_in, b_in, W_q, W_k, W_v, W_p1, b_p1, W_p2,
           b_p2, W_a1, b_a1, W_a2, b_a2, W_out, b_out):
    f32 = jnp.float32
    src0, dst0 = edge_index[0], edge_index[1]
    keep = src0 != dst0
    loop = jnp.arange(_N, dtype=src0.dtype)
    padn = _EP - _E2
    src = jnp.concatenate([jnp.where(keep, src0, _N), loop,
                           jnp.full((padn,), _N, src0.dtype)])
    dst = jnp.concatenate([jnp.where(keep, dst0, _N), loop,
                           jnp.full((padn,), _N, dst0.dtype)])
    x_pad = jnp.pad(x, ((0, _NPAD - _N), (0, 0)))
    pos_pad = jnp.pad(pos.astype(f32), ((0, _NPAD - _N), (0, 14)))
    wp1_pad = jnp.pad(W_p1, ((0, 14), (0, 0)))
    r2 = lambda v: v.reshape(1, -1)

    nb = _NPAD // _BLKN
    t1, tv = pl.pallas_call(
        _node_tables_kernel,
        grid=(nb,),
        in_specs=[
            pl.BlockSpec((_BLKN, _D), lambda i: (i, 0)),
            pl.BlockSpec((_BLKN, 16), lambda i: (i, 0)),
            _full((_D, _D)), _full((1, _D)), _full((_D, _D)),
            _full((_D, _D)), _full((_D, _D)), _full((_D, _H)),
            _full((16, _H)),
        ],
        out_specs=[
            pl.BlockSpec((_BLKN, _D), lambda i: (i, 0)),
            pl.BlockSpec((_BLKN, 2 * _D), lambda i: (i, 0)),
        ],
        out_shape=[
            jax.ShapeDtypeStruct((_NPAD, _D), f32),
            jax.ShapeDtypeStruct((_NPAD, 2 * _D), f32),
        ],
    )(x_pad, pos_pad, W_in, r2(b_in), W_q, W_k, W_v, W_a1, wp1_pad)

    g, vs = _sc_gather(t1, tv, dst, src)

    neb = _EP // _EB
    ex, m = pl.pallas_call(
        _edge_mlp_kernel,
        grid=(neb,),
        in_specs=[
            pl.BlockSpec((_EB, _D), lambda i: (i, 0)),
            pl.BlockSpec((_EB, _D), lambda i: (i, 0)),
            _full((1, _H)), _full((_H, _D)), _full((1, _D)),
            _full((_D, _H)), _full((1, _H)), _full((_H, _D)), _full((1, _D)),
        ],
        out_specs=[
            pl.BlockSpec((_EB, _D), lambda i: (i, 0)),
            pl.BlockSpec((_EB, _D), lambda i: (i, 0)),
        ],
        out_shape=[
            jax.ShapeDtypeStruct((_EP, _D), f32),
            jax.ShapeDtypeStruct((_EP, _D), f32),
        ],
    )(g, vs, r2(b_p1), W_p2, r2(b_p2), W_a1, r2(b_a1), W_a2, r2(b_a2))

    npart, dpart = _sc_scatter(m, ex, dst)

    outp = pl.pallas_call(
        _out_kernel,
        grid=(nb,),
        in_specs=[
            pl.BlockSpec((_BLKN, _D), lambda i: (i, 0)),
            pl.BlockSpec((_BLKN, _D), lambda i: (i + nb, 0)),
            pl.BlockSpec((_BLKN, _D), lambda i: (i, 0)),
            pl.BlockSpec((_BLKN, _D), lambda i: (i + nb, 0)),
            _full((_D, _D)), _full((1, _D)),
        ],
        out_specs=pl.BlockSpec((_BLKN, _D), lambda i: (i, 0)),
        out_shape=jax.ShapeDtypeStruct((_NPAD, _D), f32),
    )(npart, npart, dpart, dpart, W_out, r2(b_out))

    return outp[:_N]


# speedup vs baseline: 1.1644x; 1.1644x over previous
import functools

import jax
import jax.numpy as jnp
from jax import lax
from jax.experimental import pallas as pl
from jax.experimental.pallas import tpu as pltpu
from jax.experimental.pallas import tpu_sc as plsc

_N = 10000
_D = 128
_H = 64
_E_RAW = 320000
_E2 = _E_RAW + _N
_CH = 256
_NC, _NS = 2, 16
_NW = _NC * _NS
_EP = ((_E2 + _NW * _CH - 1) // (_NW * _CH)) * (_NW * _CH)
_EPW = _EP // _NW
_NCHUNK = _EPW // _CH
_NPAD = 10240
_RPT = _NPAD // _NS
_BLKN = 512
_EB = 4096

_sc_mesh = plsc.VectorSubcoreMesh(
    core_axis_name="c", subcore_axis_name="s", num_cores=_NC, num_subcores=_NS)


def _node_tables_kernel(x_ref, pos_ref, win_ref, bin_ref, wq_ref, wk_ref,
                        wv_ref, wa1_ref, wp1_ref, t1_ref, t2_ref, v_ref):
    x1 = jnp.maximum(
        jnp.dot(x_ref[...], win_ref[...], preferred_element_type=jnp.float32)
        + bin_ref[...], 0.0)
    qa = jnp.dot(jnp.dot(x1, wq_ref[...], preferred_element_type=jnp.float32),
                 wa1_ref[...], preferred_element_type=jnp.float32)
    ka = jnp.dot(jnp.dot(x1, wk_ref[...], preferred_element_type=jnp.float32),
                 wa1_ref[...], preferred_element_type=jnp.float32)
    pp = jnp.dot(pos_ref[...], wp1_ref[...], preferred_element_type=jnp.float32)
    t1_ref[...] = jnp.concatenate([qa, pp], axis=1)
    t2_ref[...] = jnp.concatenate([ka, pp], axis=1)
    v_ref[...] = jnp.dot(x1, wv_ref[...], preferred_element_type=jnp.float32)


def _edge_mlp_kernel(g_ref, vs_ref, bp1_ref, wp2_ref, bp2_ref, wa1_ref,
                     ba1_ref, wa2_ref, ba2_ref, ex_ref, m_ref):
    g = g_ref[...]
    hp = jnp.maximum(g[:, _H:] + bp1_ref[...], 0.0)
    delta = jnp.maximum(
        jnp.dot(hp, wp2_ref[...], preferred_element_type=jnp.float32)
        + bp2_ref[...], 0.0)
    da = jnp.dot(delta, wa1_ref[...], preferred_element_type=jnp.float32)
    h1 = jnp.maximum(g[:, :_H] + da + ba1_ref[...], 0.0)
    a = jnp.maximum(
        jnp.dot(h1, wa2_ref[...], preferred_element_type=jnp.float32)
        + ba2_ref[...], 0.0)
    ex = jnp.exp(a)
    ex_ref[...] = ex
    m_ref[...] = ex * (vs_ref[...] + delta)


def _out_kernel(nlo_ref, nhi_ref, dlo_ref, dhi_ref, wout_ref, bout_ref, o_ref):
    num = nlo_ref[...] + nhi_ref[...]
    den = dlo_ref[...] + dhi_ref[...]
    node = num / (den + 1e-16)
    o_ref[...] = jnp.maximum(
        jnp.dot(node, wout_ref[...], preferred_element_type=jnp.float32)
        + bout_ref[...], 0.0)


@functools.partial(
    pl.kernel,
    out_type=[
        jax.ShapeDtypeStruct((_EP, _D), jnp.float32),
        jax.ShapeDtypeStruct((_EP, _D), jnp.float32),
    ],
    mesh=_sc_mesh,
    scratch_types=[
        pltpu.VMEM((_CH,), jnp.int32),
        pltpu.VMEM((_CH,), jnp.int32),
        pltpu.VMEM((_CH, _D), jnp.float32),
        pltpu.VMEM((_CH, _D), jnp.float32),
        pltpu.VMEM((_CH, _D), jnp.float32),
        pltpu.SemaphoreType.DMA,
        pltpu.SemaphoreType.DMA,
        pltpu.SemaphoreType.DMA,
    ],
)
def _sc_gather(t1, t2, vtab, dstr, srcr, g, vs, di, si, b1, b2, bv,
               s1, s2, s3):
    c = lax.axis_index("c")
    s = lax.axis_index("s")
    base0 = (s * _NC + c) * _EPW

    def chunk(i, carry):
        base = base0 + i * _CH
        ha = pltpu.async_copy(dstr.at[pl.ds(base, _CH)], di, s1)
        hb = pltpu.async_copy(srcr.at[pl.ds(base, _CH)], si, s2)
        ha.wait()
        hb.wait()
        h1 = pltpu.async_copy(t1.at[di], b1, s1)
        h2 = pltpu.async_copy(t2.at[si], b2, s2)
        h3 = pltpu.async_copy(vtab.at[si], bv, s3)
        h1.wait()
        h2.wait()
        h3.wait()

        def row(r, cr):
            for cc in range(_D // 16):
                sl = pl.ds(cc * 16, 16)
                b1[r, sl] = b1[r, sl] - b2[r, sl]
            return cr

        lax.fori_loop(0, _CH, row, 0)
        hw1 = pltpu.async_copy(b1, g.at[pl.ds(base, _CH)], s1)
        hw2 = pltpu.async_copy(bv, vs.at[pl.ds(base, _CH)], s2)
        hw1.wait()
        hw2.wait()
        return carry

    lax.fori_loop(0, _NCHUNK, chunk, 0)


def _zero_acc(buf, acc, s):
    def zrow(r, cr):
        for cc in range(_D // 16):
            buf[r, pl.ds(cc * 16, 16)] = jnp.zeros((16,), jnp.float32)
        return cr

    lax.fori_loop(0, _CH, zrow, 0)
    off = 0
    for sz in (_CH, _CH, _RPT - 2 * _CH):
        pltpu.sync_copy(buf.at[pl.ds(0, sz)],
                        acc.at[pl.ds(s * _RPT + off, sz)])
        off += sz


@functools.partial(
    pl.kernel,
    out_type=[
        jax.ShapeDtypeStruct((2 * _NPAD, _D), jnp.float32),
        jax.ShapeDtypeStruct((2 * _NPAD, _D), jnp.float32),
    ],
    mesh=_sc_mesh,
    scratch_types=[
        pltpu.VMEM((_CH,), jnp.int32),
        pltpu.VMEM((_CH, _D), jnp.float32),
        pltpu.VMEM_SHARED((_NPAD, _D), jnp.float32),
        pltpu.SemaphoreType.DMA,
        pltpu.SemaphoreType.DMA,
    ],
)
def _sc_scatter(mr, exr, dstr, npart, dpart, di, bval, acc, s1, s2):
    c = lax.axis_index("c")
    s = lax.axis_index("s")
    base0 = (s * _NC + c) * _EPW

    for valr, part in ((mr, npart), (exr, dpart)):
        _zero_acc(bval, acc, s)
        plsc.subcore_barrier()

        def chunk(i, carry):
            base = base0 + i * _CH
            ha = pltpu.async_copy(dstr.at[pl.ds(base, _CH)], di, s1)
            hb = pltpu.async_copy(valr.at[pl.ds(base, _CH)], bval, s2)
            ha.wait()
            hb.wait()
            pltpu.sync_copy(bval, acc.at[di], add=True)
            return carry

        lax.fori_loop(0, _NCHUNK, chunk, 0)
        plsc.subcore_barrier()
        pltpu.sync_copy(acc.at[pl.ds(s * _RPT, _RPT)],
                        part.at[pl.ds(c * _NPAD + s * _RPT, _RPT)])
        plsc.subcore_barrier()


def _full(shape):
    return pl.BlockSpec(shape, lambda i: (0, 0))


def kernel(x, pos, edge_index, W_in, b_in, W_q, W_k, W_v, W_p1, b_p1, W_p2,
           b_p2, W_a1, b_a1, W_a2, b_a2, W_out, b_out):
    f32 = jnp.float32
    src0, dst0 = edge_index[0], edge_index[1]
    keep = src0 != dst0
    loop = jnp.arange(_N, dtype=src0.dtype)
    padn = _EP - _E2
    src = jnp.concatenate([jnp.where(keep, src0, _N), loop,
                           jnp.full((padn,), _N, src0.dtype)])
    dst = jnp.concatenate([jnp.where(keep, dst0, _N), loop,
                           jnp.full((padn,), _N, dst0.dtype)])
    x_pad = jnp.pad(x, ((0, _NPAD - _N), (0, 0)))
    pos_pad = jnp.pad(pos.astype(f32), ((0, _NPAD - _N), (0, 14)))
    wp1_pad = jnp.pad(W_p1, ((0, 14), (0, 0)))
    r2 = lambda v: v.reshape(1, -1)

    nb = _NPAD // _BLKN
    t1, t2, vtab = pl.pallas_call(
        _node_tables_kernel,
        grid=(nb,),
        in_specs=[
            pl.BlockSpec((_BLKN, _D), lambda i: (i, 0)),
            pl.BlockSpec((_BLKN, 16), lambda i: (i, 0)),
            _full((_D, _D)), _full((1, _D)), _full((_D, _D)),
            _full((_D, _D)), _full((_D, _D)), _full((_D, _H)),
            _full((16, _H)),
        ],
        out_specs=[
            pl.BlockSpec((_BLKN, _D), lambda i: (i, 0)),
            pl.BlockSpec((_BLKN, _D), lambda i: (i, 0)),
            pl.BlockSpec((_BLKN, _D), lambda i: (i, 0)),
        ],
        out_shape=[
            jax.ShapeDtypeStruct((_NPAD, _D), f32),
            jax.ShapeDtypeStruct((_NPAD, _D), f32),
            jax.ShapeDtypeStruct((_NPAD, _D), f32),
        ],
    )(x_pad, pos_pad, W_in, r2(b_in), W_q, W_k, W_v, W_a1, wp1_pad)

    g, vs = _sc_gather(t1, t2, vtab, dst, src)

    neb = _EP // _EB
    ex, m = pl.pallas_call(
        _edge_mlp_kernel,
        grid=(neb,),
        in_specs=[
            pl.BlockSpec((_EB, _D), lambda i: (i, 0)),
            pl.BlockSpec((_EB, _D), lambda i: (i, 0)),
            _full((1, _H)), _full((_H, _D)), _full((1, _D)),
            _full((_D, _H)), _full((1, _H)), _full((_H, _D)), _full((1, _D)),
        ],
        out_specs=[
            pl.BlockSpec((_EB, _D), lambda i: (i, 0)),
            pl.BlockSpec((_EB, _D), lambda i: (i, 0)),
        ],
        out_shape=[
            jax.ShapeDtypeStruct((_EP, _D), f32),
            jax.ShapeDtypeStruct((_EP, _D), f32),
        ],
    )(g, vs, r2(b_p1), W_p2, r2(b_p2), W_a1, r2(b_a1), W_a2, r2(b_a2))

    npart, dpart = _sc_scatter(m, ex, dst)

    outp = pl.pallas_call(
        _out_kernel,
        grid=(nb,),
        in_specs=[
            pl.BlockSpec((_BLKN, _D), lambda i: (i, 0)),
            pl.BlockSpec((_BLKN, _D), lambda i: (i + nb, 0)),
            pl.BlockSpec((_BLKN, _D), lambda i: (i, 0)),
            pl.BlockSpec((_BLKN, _D), lambda i: (i + nb, 0)),
            _full((_D, _D)), _full((1, _D)),
        ],
        out_specs=pl.BlockSpec((_BLKN, _D), lambda i: (i, 0)),
        out_shape=jax.ShapeDtypeStruct((_NPAD, _D), f32),
    )(npart, npart, dpart, dpart, W_out, r2(b_out))

    return outp[:_N]


# speedup vs baseline: 1.2099x; 1.0391x over previous
import functools

import jax
import jax.numpy as jnp
from jax import lax
from jax.experimental import pallas as pl
from jax.experimental.pallas import tpu as pltpu
from jax.experimental.pallas import tpu_sc as plsc

_N = 10000
_D = 128
_H = 64
_E_RAW = 320000
_E2 = _E_RAW + _N
_CH = 256
_NC, _NS = 2, 16
_NW = _NC * _NS
_EP = ((_E2 + _NW * _CH - 1) // (_NW * _CH)) * (_NW * _CH)
_EPW = _EP // _NW
_NCHUNK = _EPW // _CH
_NPAD = 10240
_RPT = _NPAD // _NS
_BLKN = 512
_EB = 4096

_sc_mesh = plsc.VectorSubcoreMesh(
    core_axis_name="c", subcore_axis_name="s", num_cores=_NC, num_subcores=_NS)


def _node_tables_kernel(x_ref, pos_ref, win_ref, bin_ref, wq_ref, wk_ref,
                        wv_ref, wa1_ref, wp1_ref, t1_ref, t2_ref, v_ref):
    x1 = jnp.maximum(
        jnp.dot(x_ref[...], win_ref[...], preferred_element_type=jnp.float32)
        + bin_ref[...], 0.0)
    qa = jnp.dot(jnp.dot(x1, wq_ref[...], preferred_element_type=jnp.float32),
                 wa1_ref[...], preferred_element_type=jnp.float32)
    ka = jnp.dot(jnp.dot(x1, wk_ref[...], preferred_element_type=jnp.float32),
                 wa1_ref[...], preferred_element_type=jnp.float32)
    pp = jnp.dot(pos_ref[...], wp1_ref[...], preferred_element_type=jnp.float32)
    t1_ref[...] = jnp.concatenate([qa, pp], axis=1)
    t2_ref[...] = jnp.concatenate([ka, pp], axis=1)
    v_ref[...] = jnp.dot(x1, wv_ref[...], preferred_element_type=jnp.float32)


def _edge_mlp_kernel(g_ref, vs_ref, bp1_ref, wp2_ref, bp2_ref, wa1_ref,
                     ba1_ref, wa2_ref, ba2_ref, ex_ref, m_ref):
    g = g_ref[...]
    hp = jnp.maximum(g[:, _H:] + bp1_ref[...], 0.0)
    delta = jnp.maximum(
        jnp.dot(hp, wp2_ref[...], preferred_element_type=jnp.float32)
        + bp2_ref[...], 0.0)
    da = jnp.dot(delta, wa1_ref[...], preferred_element_type=jnp.float32)
    h1 = jnp.maximum(g[:, :_H] + da + ba1_ref[...], 0.0)
    a = jnp.maximum(
        jnp.dot(h1, wa2_ref[...], preferred_element_type=jnp.float32)
        + ba2_ref[...], 0.0)
    ex = jnp.exp(a)
    ex_ref[...] = ex
    m_ref[...] = ex * (vs_ref[...] + delta)


def _out_kernel(nlo_ref, nhi_ref, dlo_ref, dhi_ref, wout_ref, bout_ref, o_ref):
    num = nlo_ref[...] + nhi_ref[...]
    den = dlo_ref[...] + dhi_ref[...]
    node = num / (den + 1e-16)
    o_ref[...] = jnp.maximum(
        jnp.dot(node, wout_ref[...], preferred_element_type=jnp.float32)
        + bout_ref[...], 0.0)


_CG = 128
_NPAIR = _EPW // (2 * _CG)


@functools.partial(
    pl.kernel,
    out_type=[
        jax.ShapeDtypeStruct((_EP, _D), jnp.float32),
        jax.ShapeDtypeStruct((_EP, _D), jnp.float32),
    ],
    mesh=_sc_mesh,
    scratch_types=[
        pltpu.VMEM((_CG,), jnp.int32),
        pltpu.VMEM((_CG,), jnp.int32),
        pltpu.VMEM((_CG,), jnp.int32),
        pltpu.VMEM((_CG,), jnp.int32),
        pltpu.VMEM((_CG, _D), jnp.float32),
        pltpu.VMEM((_CG, _D), jnp.float32),
        pltpu.VMEM((_CG, _D), jnp.float32),
        pltpu.VMEM((_CG, _D), jnp.float32),
        pltpu.VMEM((_CG, _D), jnp.float32),
        pltpu.VMEM((_CG, _D), jnp.float32),
    ] + [pltpu.SemaphoreType.DMA] * 10,
)
def _sc_gather(t1, t2, vtab, dstr, srcr, g, vs,
               dia, sia, dib, sib, b1a, b2a, bva, b1b, b2b, bvb,
               sa1, sa2, sa3, sb1, sb2, sb3, swa1, swa2, swb1, swb2):
    c = lax.axis_index("c")
    s = lax.axis_index("s")
    base0 = (s * _NC + c) * _EPW

    def load_idx(base, dib_, sib_):
        pltpu.sync_copy(dstr.at[pl.ds(base, _CG)], dib_)
        pltpu.sync_copy(srcr.at[pl.ds(base, _CG)], sib_)

    def sub_rows(bx, by):
        def row(r, cr):
            for cc in range(_D // 16):
                sl = pl.ds(cc * 16, 16)
                bx[r, sl] = bx[r, sl] - by[r, sl]
            return cr

        lax.fori_loop(0, _CG, row, 0)

    load_idx(base0, dia, sia)
    pltpu.async_copy(t1.at[dia], b1a, sa1)
    pltpu.async_copy(t2.at[sia], b2a, sa2)
    pltpu.async_copy(vtab.at[sia], bva, sa3)
    load_idx(base0 + _CG, dib, sib)

    def pair(i, carry):
        base_a = base0 + (2 * i) * _CG
        base_b = base_a + _CG
        hb1 = pltpu.async_copy(t1.at[dib], b1b, sb1)
        hb2 = pltpu.async_copy(t2.at[sib], b2b, sb2)
        hb3 = pltpu.async_copy(vtab.at[sib], bvb, sb3)
        pltpu.make_async_copy(t1.at[dia], b1a, sa1).wait()
        pltpu.make_async_copy(t2.at[sia], b2a, sa2).wait()
        pltpu.make_async_copy(vtab.at[sia], bva, sa3).wait()

        @pl.when(i < _NPAIR - 1)
        def _():
            load_idx(base_a + 2 * _CG, dia, sia)

        sub_rows(b1a, b2a)
        hwa1 = pltpu.async_copy(b1a, g.at[pl.ds(base_a, _CG)], swa1)
        hwa2 = pltpu.async_copy(bva, vs.at[pl.ds(base_a, _CG)], swa2)
        hb1.wait()
        hb2.wait()
        hb3.wait()

        @pl.when(i < _NPAIR - 1)
        def _():
            load_idx(base_b + 2 * _CG, dib, sib)

        sub_rows(b1b, b2b)
        hwb1 = pltpu.async_copy(b1b, g.at[pl.ds(base_b, _CG)], swb1)
        hwb2 = pltpu.async_copy(bvb, vs.at[pl.ds(base_b, _CG)], swb2)
        hwa1.wait()
        hwa2.wait()

        @pl.when(i < _NPAIR - 1)
        def _():
            pltpu.async_copy(t1.at[dia], b1a, sa1)
            pltpu.async_copy(t2.at[sia], b2a, sa2)
            pltpu.async_copy(vtab.at[sia], bva, sa3)

        hwb1.wait()
        hwb2.wait()
        return carry

    lax.fori_loop(0, _NPAIR, pair, 0)


def _zero_acc(buf, acc, s):
    def zrow(r, cr):
        for cc in range(_D // 16):
            buf[r, pl.ds(cc * 16, 16)] = jnp.zeros((16,), jnp.float32)
        return cr

    lax.fori_loop(0, _CH, zrow, 0)
    off = 0
    for sz in (_CH, _CH, _RPT - 2 * _CH):
        pltpu.sync_copy(buf.at[pl.ds(0, sz)],
                        acc.at[pl.ds(s * _RPT + off, sz)])
        off += sz


@functools.partial(
    pl.kernel,
    out_type=[
        jax.ShapeDtypeStruct((2 * _NPAD, _D), jnp.float32),
        jax.ShapeDtypeStruct((2 * _NPAD, _D), jnp.float32),
    ],
    mesh=_sc_mesh,
    scratch_types=[
        pltpu.VMEM((_CH,), jnp.int32),
        pltpu.VMEM((_CH, _D), jnp.float32),
        pltpu.VMEM_SHARED((_NPAD, _D), jnp.float32),
        pltpu.SemaphoreType.DMA,
        pltpu.SemaphoreType.DMA,
    ],
)
def _sc_scatter(mr, exr, dstr, npart, dpart, di, bval, acc, s1, s2):
    c = lax.axis_index("c")
    s = lax.axis_index("s")
    base0 = (s * _NC + c) * _EPW

    for valr, part in ((mr, npart), (exr, dpart)):
        _zero_acc(bval, acc, s)
        plsc.subcore_barrier()

        def chunk(i, carry):
            base = base0 + i * _CH
            ha = pltpu.async_copy(dstr.at[pl.ds(base, _CH)], di, s1)
            hb = pltpu.async_copy(valr.at[pl.ds(base, _CH)], bval, s2)
            ha.wait()
            hb.wait()
            pltpu.sync_copy(bval, acc.at[di], add=True)
            return carry

        lax.fori_loop(0, _NCHUNK, chunk, 0)
        plsc.subcore_barrier()
        pltpu.sync_copy(acc.at[pl.ds(s * _RPT, _RPT)],
                        part.at[pl.ds(c * _NPAD + s * _RPT, _RPT)])
        plsc.subcore_barrier()


def _full(shape):
    return pl.BlockSpec(shape, lambda i: (0, 0))


def kernel(x, pos, edge_index, W_in, b_in, W_q, W_k, W_v, W_p1, b_p1, W_p2,
           b_p2, W_a1, b_a1, W_a2, b_a2, W_out, b_out):
    f32 = jnp.float32
    src0, dst0 = edge_index[0], edge_index[1]
    keep = src0 != dst0
    loop = jnp.arange(_N, dtype=src0.dtype)
    padn = _EP - _E2
    src = jnp.concatenate([jnp.where(keep, src0, _N), loop,
                           jnp.full((padn,), _N, src0.dtype)])
    dst = jnp.concatenate([jnp.where(keep, dst0, _N), loop,
                           jnp.full((padn,), _N, dst0.dtype)])
    x_pad = jnp.pad(x, ((0, _NPAD - _N), (0, 0)))
    pos_pad = jnp.pad(pos.astype(f32), ((0, _NPAD - _N), (0, 14)))
    wp1_pad = jnp.pad(W_p1, ((0, 14), (0, 0)))
    r2 = lambda v: v.reshape(1, -1)

    nb = _NPAD // _BLKN
    t1, t2, vtab = pl.pallas_call(
        _node_tables_kernel,
        grid=(nb,),
        in_specs=[
            pl.BlockSpec((_BLKN, _D), lambda i: (i, 0)),
            pl.BlockSpec((_BLKN, 16), lambda i: (i, 0)),
            _full((_D, _D)), _full((1, _D)), _full((_D, _D)),
            _full((_D, _D)), _full((_D, _D)), _full((_D, _H)),
            _full((16, _H)),
        ],
        out_specs=[
            pl.BlockSpec((_BLKN, _D), lambda i: (i, 0)),
            pl.BlockSpec((_BLKN, _D), lambda i: (i, 0)),
            pl.BlockSpec((_BLKN, _D), lambda i: (i, 0)),
        ],
        out_shape=[
            jax.ShapeDtypeStruct((_NPAD, _D), f32),
            jax.ShapeDtypeStruct((_NPAD, _D), f32),
            jax.ShapeDtypeStruct((_NPAD, _D), f32),
        ],
    )(x_pad, pos_pad, W_in, r2(b_in), W_q, W_k, W_v, W_a1, wp1_pad)

    g, vs = _sc_gather(t1, t2, vtab, dst, src)

    neb = _EP // _EB
    ex, m = pl.pallas_call(
        _edge_mlp_kernel,
        grid=(neb,),
        in_specs=[
            pl.BlockSpec((_EB, _D), lambda i: (i, 0)),
            pl.BlockSpec((_EB, _D), lambda i: (i, 0)),
            _full((1, _H)), _full((_H, _D)), _full((1, _D)),
            _full((_D, _H)), _full((1, _H)), _full((_H, _D)), _full((1, _D)),
        ],
        out_specs=[
            pl.BlockSpec((_EB, _D), lambda i: (i, 0)),
            pl.BlockSpec((_EB, _D), lambda i: (i, 0)),
        ],
        out_shape=[
            jax.ShapeDtypeStruct((_EP, _D), f32),
            jax.ShapeDtypeStruct((_EP, _D), f32),
        ],
    )(g, vs, r2(b_p1), W_p2, r2(b_p2), W_a1, r2(b_a1), W_a2, r2(b_a2))

    npart, dpart = _sc_scatter(m, ex, dst)

    outp = pl.pallas_call(
        _out_kernel,
        grid=(nb,),
        in_specs=[
            pl.BlockSpec((_BLKN, _D), lambda i: (i, 0)),
            pl.BlockSpec((_BLKN, _D), lambda i: (i + nb, 0)),
            pl.BlockSpec((_BLKN, _D), lambda i: (i, 0)),
            pl.BlockSpec((_BLKN, _D), lambda i: (i + nb, 0)),
            _full((_D, _D)), _full((1, _D)),
        ],
        out_specs=pl.BlockSpec((_BLKN, _D), lambda i: (i, 0)),
        out_shape=jax.ShapeDtypeStruct((_NPAD, _D), f32),
    )(npart, npart, dpart, dpart, W_out, r2(b_out))

    return outp[:_N]


# speedup vs baseline: 1.2108x; 1.0008x over previous
import functools

import jax
import jax.numpy as jnp
from jax import lax
from jax.experimental import pallas as pl
from jax.experimental.pallas import tpu as pltpu
from jax.experimental.pallas import tpu_sc as plsc

_N = 10000
_D = 128
_H = 64
_E_RAW = 320000
_E2 = _E_RAW + _N
_CH = 256
_NC, _NS = 2, 16
_NW = _NC * _NS
_EP = ((_E2 + _NW * _CH - 1) // (_NW * _CH)) * (_NW * _CH)
_EPW = _EP // _NW
_NCHUNK = _EPW // _CH
_NPAD = 10240
_RPT = _NPAD // _NS
_BLKN = 512
_EB = 4096

_sc_mesh = plsc.VectorSubcoreMesh(
    core_axis_name="c", subcore_axis_name="s", num_cores=_NC, num_subcores=_NS)


_DG = _D


def _node_tables_kernel(x_ref, pos_ref, win_ref, bin_ref, wq_ref, wk_ref,
                        wv_ref, wa1_ref, wp1_ref, t1_ref, t2_ref, v_ref):
    x1 = jnp.maximum(
        jnp.dot(x_ref[...], win_ref[...], preferred_element_type=jnp.float32)
        + bin_ref[...], 0.0)
    qa = jnp.dot(jnp.dot(x1, wq_ref[...], preferred_element_type=jnp.float32),
                 wa1_ref[...], preferred_element_type=jnp.float32)
    ka = jnp.dot(jnp.dot(x1, wk_ref[...], preferred_element_type=jnp.float32),
                 wa1_ref[...], preferred_element_type=jnp.float32)
    pp = jnp.dot(pos_ref[...], wp1_ref[...], preferred_element_type=jnp.float32)
    t1_ref[...] = jnp.concatenate([qa, pp], axis=1)
    t2_ref[...] = jnp.concatenate([ka, pp], axis=1)
    v_ref[...] = jnp.dot(x1, wv_ref[...], preferred_element_type=jnp.float32)


def _edge_mlp_kernel(g_ref, vs_ref, bp1_ref, wp2_ref, bp2_ref, wa1_ref,
                     ba1_ref, wa2_ref, ba2_ref, ex_ref, m_ref):
    g = g_ref[...]
    hp = jnp.maximum(g[:, _H:] + bp1_ref[...], 0.0)
    delta = jnp.maximum(
        jnp.dot(hp, wp2_ref[...], preferred_element_type=jnp.float32)
        + bp2_ref[...], 0.0)
    da = jnp.dot(delta, wa1_ref[...], preferred_element_type=jnp.float32)
    h1 = jnp.maximum(g[:, :_H] + da + ba1_ref[...], 0.0)
    a = jnp.maximum(
        jnp.dot(h1, wa2_ref[...], preferred_element_type=jnp.float32)
        + ba2_ref[...], 0.0)
    ex = jnp.exp(a)
    ex_ref[...] = ex
    m_ref[...] = ex * (vs_ref[...] + delta)


def _out_kernel(nlo_ref, nhi_ref, dlo_ref, dhi_ref, wout_ref, bout_ref, o_ref):
    num = nlo_ref[...] + nhi_ref[...]
    den = dlo_ref[...] + dhi_ref[...]
    node = num / (den + 1e-16)
    o_ref[...] = jnp.maximum(
        jnp.dot(node, wout_ref[...], preferred_element_type=jnp.float32)
        + bout_ref[...], 0.0)


_CG = 128
_NPAIR = _EPW // (2 * _CG)


@functools.partial(
    pl.kernel,
    out_type=[
        jax.ShapeDtypeStruct((_EP, _DG), jnp.float32),
        jax.ShapeDtypeStruct((_EP, _D), jnp.float32),
    ],
    mesh=_sc_mesh,
    scratch_types=[
        pltpu.VMEM((_CG,), jnp.int32),
        pltpu.VMEM((_CG,), jnp.int32),
        pltpu.VMEM((_CG,), jnp.int32),
        pltpu.VMEM((_CG,), jnp.int32),
        pltpu.VMEM((_CG, _DG), jnp.float32),
        pltpu.VMEM((_CG, _DG), jnp.float32),
        pltpu.VMEM((_CG, _D), jnp.float32),
        pltpu.VMEM((_CG, _DG), jnp.float32),
        pltpu.VMEM((_CG, _DG), jnp.float32),
        pltpu.VMEM((_CG, _D), jnp.float32),
    ] + [pltpu.SemaphoreType.DMA] * 10,
)
def _sc_gather(t1, t2, vtab, dstr, srcr, g, vs,
               dia, sia, dib, sib, b1a, b2a, bva, b1b, b2b, bvb,
               sa1, sa2, sa3, sb1, sb2, sb3, swa1, swa2, swb1, swb2):
    c = lax.axis_index("c")
    s = lax.axis_index("s")
    base0 = (s * _NC + c) * _EPW

    def load_idx(base, dib_, sib_):
        pltpu.sync_copy(dstr.at[pl.ds(base, _CG)], dib_)
        pltpu.sync_copy(srcr.at[pl.ds(base, _CG)], sib_)

    def sub_rows(bx, by):
        def row(r, cr):
            for cc in range(_DG // 16):
                sl = pl.ds(cc * 16, 16)
                bx[r, sl] = bx[r, sl] - by[r, sl]
            return cr

        lax.fori_loop(0, _CG, row, 0)

    load_idx(base0, dia, sia)
    pltpu.async_copy(t1.at[dia], b1a, sa1)
    pltpu.async_copy(t2.at[sia], b2a, sa2)
    pltpu.async_copy(vtab.at[sia], bva, sa3)
    load_idx(base0 + _CG, dib, sib)

    def pair(i, carry):
        base_a = base0 + (2 * i) * _CG
        base_b = base_a + _CG
        hb1 = pltpu.async_copy(t1.at[dib], b1b, sb1)
        hb2 = pltpu.async_copy(t2.at[sib], b2b, sb2)
        hb3 = pltpu.async_copy(vtab.at[sib], bvb, sb3)
        pltpu.make_async_copy(t1.at[dia], b1a, sa1).wait()
        pltpu.make_async_copy(t2.at[sia], b2a, sa2).wait()
        pltpu.make_async_copy(vtab.at[sia], bva, sa3).wait()

        @pl.when(i < _NPAIR - 1)
        def _():
            load_idx(base_a + 2 * _CG, dia, sia)

        sub_rows(b1a, b2a)
        hwa1 = pltpu.async_copy(b1a, g.at[pl.ds(base_a, _CG)], swa1)
        hwa2 = pltpu.async_copy(bva, vs.at[pl.ds(base_a, _CG)], swa2)
        hb1.wait()
        hb2.wait()
        hb3.wait()

        @pl.when(i < _NPAIR - 1)
        def _():
            load_idx(base_b + 2 * _CG, dib, sib)

        sub_rows(b1b, b2b)
        hwb1 = pltpu.async_copy(b1b, g.at[pl.ds(base_b, _CG)], swb1)
        hwb2 = pltpu.async_copy(bvb, vs.at[pl.ds(base_b, _CG)], swb2)
        hwa1.wait()
        hwa2.wait()

        @pl.when(i < _NPAIR - 1)
        def _():
            pltpu.async_copy(t1.at[dia], b1a, sa1)
            pltpu.async_copy(t2.at[sia], b2a, sa2)
            pltpu.async_copy(vtab.at[sia], bva, sa3)

        hwb1.wait()
        hwb2.wait()
        return carry

    lax.fori_loop(0, _NPAIR, pair, 0)


def _zero_acc(buf, acc, s):
    def zrow(r, cr):
        for cc in range(_D // 16):
            buf[r, pl.ds(cc * 16, 16)] = jnp.zeros((16,), jnp.float32)
        return cr

    lax.fori_loop(0, _CH, zrow, 0)
    off = 0
    for sz in (_CH, _CH, _RPT - 2 * _CH):
        pltpu.sync_copy(buf.at[pl.ds(0, sz)],
                        acc.at[pl.ds(s * _RPT + off, sz)])
        off += sz


@functools.partial(
    pl.kernel,
    out_type=[
        jax.ShapeDtypeStruct((2 * _NPAD, _D), jnp.float32),
        jax.ShapeDtypeStruct((2 * _NPAD, _D), jnp.float32),
    ],
    mesh=_sc_mesh,
    scratch_types=[
        pltpu.VMEM((_CH,), jnp.int32),
        pltpu.VMEM((_CH, _D), jnp.float32),
        pltpu.VMEM_SHARED((_NPAD, _D), jnp.float32),
        pltpu.SemaphoreType.DMA,
        pltpu.SemaphoreType.DMA,
    ],
)
def _sc_scatter(mr, exr, dstr, npart, dpart, di, bval, acc, s1, s2):
    c = lax.axis_index("c")
    s = lax.axis_index("s")
    base0 = (s * _NC + c) * _EPW

    for valr, part in ((mr, npart), (exr, dpart)):
        _zero_acc(bval, acc, s)
        plsc.subcore_barrier()

        def chunk(i, carry):
            base = base0 + i * _CH
            ha = pltpu.async_copy(dstr.at[pl.ds(base, _CH)], di, s1)
            hb = pltpu.async_copy(valr.at[pl.ds(base, _CH)], bval, s2)
            ha.wait()
            hb.wait()
            pltpu.sync_copy(bval, acc.at[di], add=True)
            return carry

        lax.fori_loop(0, _NCHUNK, chunk, 0)
        plsc.subcore_barrier()
        pltpu.sync_copy(acc.at[pl.ds(s * _RPT, _RPT)],
                        part.at[pl.ds(c * _NPAD + s * _RPT, _RPT)])
        plsc.subcore_barrier()


def _full(shape):
    return pl.BlockSpec(shape, lambda i: (0, 0))


def kernel(x, pos, edge_index, W_in, b_in, W_q, W_k, W_v, W_p1, b_p1, W_p2,
           b_p2, W_a1, b_a1, W_a2, b_a2, W_out, b_out):
    f32 = jnp.float32
    src0, dst0 = edge_index[0], edge_index[1]
    keep = src0 != dst0
    loop = jnp.arange(_N, dtype=src0.dtype)
    padn = _EP - _E2
    src = jnp.concatenate([jnp.where(keep, src0, _N), loop,
                           jnp.full((padn,), _N, src0.dtype)])
    dst = jnp.concatenate([jnp.where(keep, dst0, _N), loop,
                           jnp.full((padn,), _N, dst0.dtype)])
    x_pad = jnp.pad(x, ((0, _NPAD - _N), (0, 0)))
    pos_pad = jnp.pad(pos.astype(f32), ((0, _NPAD - _N), (0, 14)))
    wp1_pad = jnp.pad(W_p1, ((0, 14), (0, 0)))
    r2 = lambda v: v.reshape(1, -1)

    nb = _NPAD // _BLKN
    t1, t2, vtab = pl.pallas_call(
        _node_tables_kernel,
        grid=(nb,),
        in_specs=[
            pl.BlockSpec((_BLKN, _D), lambda i: (i, 0)),
            pl.BlockSpec((_BLKN, 16), lambda i: (i, 0)),
            _full((_D, _D)), _full((1, _D)), _full((_D, _D)),
            _full((_D, _D)), _full((_D, _D)), _full((_D, _H)),
            _full((16, _H)),
        ],
        out_specs=[
            pl.BlockSpec((_BLKN, _DG), lambda i: (i, 0)),
            pl.BlockSpec((_BLKN, _DG), lambda i: (i, 0)),
            pl.BlockSpec((_BLKN, _D), lambda i: (i, 0)),
        ],
        out_shape=[
            jax.ShapeDtypeStruct((_NPAD, _DG), f32),
            jax.ShapeDtypeStruct((_NPAD, _DG), f32),
            jax.ShapeDtypeStruct((_NPAD, _D), f32),
        ],
    )(x_pad, pos_pad, W_in, r2(b_in), W_q, W_k, W_v, W_a1, wp1_pad)

    g, vs = _sc_gather(t1, t2, vtab, dst, src)

    neb = _EP // _EB
    ex, m = pl.pallas_call(
        _edge_mlp_kernel,
        grid=(neb,),
        in_specs=[
            pl.BlockSpec((_EB, _DG), lambda i: (i, 0)),
            pl.BlockSpec((_EB, _D), lambda i: (i, 0)),
            _full((1, _H)), _full((_H, _D)), _full((1, _D)),
            _full((_D, _H)), _full((1, _H)), _full((_H, _D)), _full((1, _D)),
        ],
        out_specs=[
            pl.BlockSpec((_EB, _D), lambda i: (i, 0)),
            pl.BlockSpec((_EB, _D), lambda i: (i, 0)),
        ],
        out_shape=[
            jax.ShapeDtypeStruct((_EP, _D), f32),
            jax.ShapeDtypeStruct((_EP, _D), f32),
        ],
    )(g, vs, r2(b_p1), W_p2, r2(b_p2), W_a1, r2(b_a1), W_a2, r2(b_a2))

    npart, dpart = _sc_scatter(m, ex, dst)

    outp = pl.pallas_call(
        _out_kernel,
        grid=(nb,),
        in_specs=[
            pl.BlockSpec((_BLKN, _D), lambda i: (i, 0)),
            pl.BlockSpec((_BLKN, _D), lambda i: (i + nb, 0)),
            pl.BlockSpec((_BLKN, _D), lambda i: (i, 0)),
            pl.BlockSpec((_BLKN, _D), lambda i: (i + nb, 0)),
            _full((_D, _D)), _full((1, _D)),
        ],
        out_specs=pl.BlockSpec((_BLKN, _D), lambda i: (i, 0)),
        out_shape=jax.ShapeDtypeStruct((_NPAD, _D), f32),
    )(npart, npart, dpart, dpart, W_out, r2(b_out))

    return outp[:_N]
